# baseline (speedup 1.0000x reference)
import jax
import jax.numpy as jnp
from jax.experimental import pallas as pl
from jax.experimental.pallas import tpu as pltpu

_B = 32768
_NC = 5
_CF = 64
_CD = 64
_SD = 128

_BB = 8192

_DN1 = (((0,), (0,)), ((), ()))
_DN2 = (((0,), (0,)), ((), ()))


def _fused_kernel(ct_ref, mask_ref, wmap_ref, bmap_ref, wred_ref,
                  bred_ref, out_ref, acc_ref):
    j = pl.program_id(1)
    x = ct_ref[0]
    h = jnp.maximum(
        jax.lax.dot_general(wmap_ref[...], x, _DN1,
                            preferred_element_type=jnp.float32)
        + bmap_ref[...], 0.0)
    h = h * mask_ref[0].astype(jnp.float32)
    p = jax.lax.dot_general(wred_ref[0], h, _DN2,
                            preferred_element_type=jnp.float32)

    @pl.when(j == 0)
    def _():
        acc_ref[...] = p

    @pl.when(j != 0)
    def _():
        acc_ref[...] = acc_ref[...] + p

    @pl.when(j == _NC - 1)
    def _():
        out_ref[...] = jnp.maximum(acc_ref[...] + bred_ref[...], 0.0).T


@jax.jit
def kernel(counts, observed_mask, W_map, b_map, W_red, b_red):
    ct = jnp.transpose(counts, (1, 2, 0))
    mt = observed_mask.T.reshape(_NC, 1, _B)
    wred3 = W_red.reshape(_NC, _CD, _SD)
    grid = (_B // _BB, _NC)
    outT = pl.pallas_call(
        _fused_kernel,
        grid=grid,
        in_specs=[
            pl.BlockSpec((1, _CF, _BB), lambda i, j: (j, 0, i)),
            pl.BlockSpec((1, 1, _BB), lambda i, j: (j, 0, i)),
            pl.BlockSpec((_CF, _CD), lambda i, j: (0, 0)),
            pl.BlockSpec((_CD, 1), lambda i, j: (0, 0)),
            pl.BlockSpec((1, _CD, _SD), lambda i, j: (j, 0, 0)),
            pl.BlockSpec((_SD, 1), lambda i, j: (0, 0)),
        ],
        out_specs=pl.BlockSpec((_BB, _SD), lambda i, j: (i, 0)),
        out_shape=jax.ShapeDtypeStruct((_B, _SD), jnp.float32),
        scratch_shapes=[pltpu.VMEM((_SD, _BB), jnp.float32)],
        compiler_params=pltpu.CompilerParams(
            dimension_semantics=("parallel", "arbitrary")),
    )(ct, mt, W_map, b_map.reshape(_CD, 1), wred3, b_red.reshape(_SD, 1))
    return outT
